# baseline (speedup 1.0000x reference)
import functools

import jax
import jax.numpy as jnp
from jax import lax
from jax.experimental import pallas as pl
from jax.experimental.pallas import tpu as pltpu
from jax.experimental.pallas import tpu_sc as plsc

_L = 16


def _fast_rsqrt(x):
    i = lax.bitcast_convert_type(x, jnp.int32)
    i = jnp.int32(0x5F3759DF) - lax.shift_right_arithmetic(i, 1)
    y = lax.bitcast_convert_type(i, jnp.float32)
    for _ in range(2):
        y = y * (1.5 - 0.5 * x * y * y)
    return y


def _make_sc_kernel(B, T, C, K):
    assert C % _L == 0
    n_groups = T // _L
    c_chunks = C // _L
    mesh = plsc.VectorSubcoreMesh(
        core_axis_name="c", subcore_axis_name="s", num_cores=2, num_subcores=16
    )

    @functools.partial(
        pl.kernel,
        out_type=jax.ShapeDtypeStruct((B, C), jnp.float32),
        mesh=mesh,
        scratch_types=[
            pltpu.VMEM((T, C), jnp.float32),
            pltpu.VMEM((K, T), jnp.int32),
            pltpu.VMEM((T,), jnp.float32),
            pltpu.VMEM((T,), jnp.float32),
            pltpu.VMEM((K * _L, _L + 1), jnp.float32),
            pltpu.VMEM((C,), jnp.float32),
        ],
        compiler_params=pltpu.CompilerParams(
            needs_layout_passes=False, use_tc_tiling_on_sc=False
        ),
    )
    def sc_kernel(x_hbm, idx_hbm, out_hbm, x_v, idx_v, n2_v, w_v, red_v, o_v):
        cid = lax.axis_index("c")
        sid = lax.axis_index("s")
        b = sid * 2 + cid

        pltpu.sync_copy(x_hbm.at[b], x_v)
        pltpu.sync_copy(idx_hbm, idx_v)

        zero16 = jnp.zeros((_L,), jnp.float32)
        lane = lax.iota(jnp.int32, _L)

        def norm_body(g, carry):
            def row_body(sl, c2):
                s = g * _L + sl
                acc = x_v[s, pl.ds(0, _L)] * x_v[s, pl.ds(0, _L)]
                for k in range(1, c_chunks):
                    v = x_v[s, pl.ds(k * _L, _L)]
                    acc = acc + v * v
                red_v[sl, pl.ds(0, _L)] = acc
                return c2

            lax.fori_loop(0, _L, row_body, 0, unroll=4)
            n2 = zero16
            for c in range(_L):
                n2 = n2 + plsc.load_gather(
                    red_v, [lane, jnp.full((_L,), c, jnp.int32)]
                )
            n2_v[pl.ds(g * _L, _L)] = n2
            w_v[pl.ds(g * _L, _L)] = zero16
            return carry

        lax.fori_loop(0, n_groups, norm_body, 0)

        def group_body(g, carry):
            t0 = g * _L
            idxv = [idx_v[j, pl.ds(t0, _L)] for j in range(K)]

            for tl in range(_L):
                t = t0 + tl
                q = [x_v[t, pl.ds(k * _L, _L)] for k in range(c_chunks)]
                for j in range(K):
                    s = idxv[j][tl]
                    acc = q[0] * x_v[s, pl.ds(0, _L)]
                    for k in range(1, c_chunks):
                        acc = acc + q[k] * x_v[s, pl.ds(k * _L, _L)]
                    red_v[j * _L + tl, pl.ds(0, _L)] = acc

            dots = []
            for j in range(K):
                rows = j * _L + lane
                d = plsc.load_gather(red_v, [rows, jnp.zeros((_L,), jnp.int32)])
                for c in range(1, _L):
                    d = d + plsc.load_gather(
                        red_v, [rows, jnp.full((_L,), c, jnp.int32)]
                    )
                dots.append(d)

            q2 = n2_v[pl.ds(t0, _L)]
            n2s = [plsc.load_gather(n2_v, [idxv[j]]) for j in range(K)]
            sims = [
                dots[j] * _fast_rsqrt(jnp.maximum(q2 * n2s[j], 1e-16))
                for j in range(K)
            ]
            m = sims[0]
            for j in range(1, K):
                m = jnp.maximum(m, sims[j])
            exps = [jnp.exp(s - m) for s in sims]
            tot = exps[0]
            for j in range(1, K):
                tot = tot + exps[j]
            r = 1.0 / tot
            for j in range(K):
                plsc.addupdate_scatter(w_v, [idxv[j]], exps[j] * r)
            return carry

        lax.fori_loop(0, n_groups, group_body, 0)

        def wrow_body(g, acc):
            w16 = w_v[pl.ds(g * _L, _L)]
            for sl in range(_L):
                s = g * _L + sl
                wv = jnp.full((_L,), w16[sl])
                acc = tuple(
                    acc[k] + wv * x_v[s, pl.ds(k * _L, _L)]
                    for k in range(c_chunks)
                )
            return acc

        acc = lax.fori_loop(
            0, n_groups, wrow_body, tuple(zero16 for _ in range(c_chunks))
        )
        scale = jnp.float32(1.0 / T)
        for k in range(c_chunks):
            o_v[pl.ds(k * _L, _L)] = acc[k] * scale
        pltpu.sync_copy(o_v, out_hbm.at[b])

    return sc_kernel


def kernel(X, neighbor_idx):
    B, T, C = X.shape
    K = neighbor_idx.shape[1]
    idx32 = neighbor_idx.astype(jnp.int32)
    out = _make_sc_kernel(B, T, C, K)(X, idx32.T)
    return out.reshape(B, 1, C)


# speedup vs baseline: 1.0036x; 1.0036x over previous
import functools

import jax
import jax.numpy as jnp
from jax import lax
from jax.experimental import pallas as pl
from jax.experimental.pallas import tpu as pltpu
from jax.experimental.pallas import tpu_sc as plsc

_L = 16


def _fast_rsqrt(x):
    i = lax.bitcast_convert_type(x, jnp.int32)
    i = jnp.int32(0x5F3759DF) - lax.shift_right_arithmetic(i, 1)
    y = lax.bitcast_convert_type(i, jnp.float32)
    for _ in range(2):
        y = y * (1.5 - 0.5 * x * y * y)
    return y


def _make_sc_kernel(B, T, C, K):
    assert C % _L == 0
    n_groups = T // _L
    c_chunks = C // _L
    mesh = plsc.VectorSubcoreMesh(
        core_axis_name="c", subcore_axis_name="s", num_cores=2, num_subcores=16
    )

    @functools.partial(
        pl.kernel,
        out_type=jax.ShapeDtypeStruct((B, C), jnp.float32),
        mesh=mesh,
        scratch_types=[
            pltpu.VMEM((T, C), jnp.float32),
            pltpu.VMEM((K, T), jnp.int32),
            pltpu.VMEM((T,), jnp.float32),
            pltpu.VMEM((T,), jnp.float32),
            pltpu.VMEM((K * _L, _L + 1), jnp.float32),
            pltpu.VMEM((C,), jnp.float32),
        ],
        compiler_params=pltpu.CompilerParams(
            needs_layout_passes=False, use_tc_tiling_on_sc=False
        ),
    )
    def sc_kernel(x_hbm, idx_hbm, out_hbm, x_v, idx_v, n2_v, w_v, red_v, o_v):
        cid = lax.axis_index("c")
        sid = lax.axis_index("s")
        b = sid * 2 + cid

        pltpu.sync_copy(x_hbm.at[b], x_v)
        pltpu.sync_copy(idx_hbm, idx_v)
        plsc.subcore_barrier()

        zero16 = jnp.zeros((_L,), jnp.float32)
        lane = lax.iota(jnp.int32, _L)

        def norm_body(g, carry):
            def row_body(sl, c2):
                s = g * _L + sl
                acc = x_v[s, pl.ds(0, _L)] * x_v[s, pl.ds(0, _L)]
                for k in range(1, c_chunks):
                    v = x_v[s, pl.ds(k * _L, _L)]
                    acc = acc + v * v
                red_v[sl, pl.ds(0, _L)] = acc
                return c2

            lax.fori_loop(0, _L, row_body, 0, unroll=4)
            n2 = zero16
            for c in range(_L):
                n2 = n2 + plsc.load_gather(
                    red_v, [lane, jnp.full((_L,), c, jnp.int32)]
                )
            n2_v[pl.ds(g * _L, _L)] = n2
            w_v[pl.ds(g * _L, _L)] = zero16
            return carry

        lax.fori_loop(0, n_groups, norm_body, 0)
        plsc.subcore_barrier()

        def group_body(g, carry):
            t0 = g * _L
            idxv = [idx_v[j, pl.ds(t0, _L)] for j in range(K)]

            for tl in range(_L):
                t = t0 + tl
                q = [x_v[t, pl.ds(k * _L, _L)] for k in range(c_chunks)]
                for j in range(K):
                    s = idxv[j][tl]
                    acc = q[0] * x_v[s, pl.ds(0, _L)]
                    for k in range(1, c_chunks):
                        acc = acc + q[k] * x_v[s, pl.ds(k * _L, _L)]
                    red_v[j * _L + tl, pl.ds(0, _L)] = acc

            dots = []
            for j in range(K):
                rows = j * _L + lane
                d = plsc.load_gather(red_v, [rows, jnp.zeros((_L,), jnp.int32)])
                for c in range(1, _L):
                    d = d + plsc.load_gather(
                        red_v, [rows, jnp.full((_L,), c, jnp.int32)]
                    )
                dots.append(d)

            q2 = n2_v[pl.ds(t0, _L)]
            n2s = [plsc.load_gather(n2_v, [idxv[j]]) for j in range(K)]
            sims = [
                dots[j] * _fast_rsqrt(jnp.maximum(q2 * n2s[j], 1e-16))
                for j in range(K)
            ]
            m = sims[0]
            for j in range(1, K):
                m = jnp.maximum(m, sims[j])
            exps = [jnp.exp(s - m) for s in sims]
            tot = exps[0]
            for j in range(1, K):
                tot = tot + exps[j]
            r = 1.0 / tot
            for j in range(K):
                plsc.addupdate_scatter(w_v, [idxv[j]], exps[j] * r)
            return carry

        lax.fori_loop(0, n_groups, group_body, 0)

        def wrow_body(g, acc):
            w16 = w_v[pl.ds(g * _L, _L)]
            for sl in range(_L):
                s = g * _L + sl
                wv = jnp.full((_L,), w16[sl])
                acc = tuple(
                    acc[k] + wv * x_v[s, pl.ds(k * _L, _L)]
                    for k in range(c_chunks)
                )
            return acc

        acc = lax.fori_loop(
            0, n_groups, wrow_body, tuple(zero16 for _ in range(c_chunks))
        )
        scale = jnp.float32(1.0 / T)
        for k in range(c_chunks):
            o_v[pl.ds(k * _L, _L)] = acc[k] * scale
        pltpu.sync_copy(o_v, out_hbm.at[b])

    return sc_kernel


def kernel(X, neighbor_idx):
    B, T, C = X.shape
    K = neighbor_idx.shape[1]
    idx32 = neighbor_idx.astype(jnp.int32)
    out = _make_sc_kernel(B, T, C, K)(X, idx32.T)
    return out.reshape(B, 1, C)


# speedup vs baseline: 2.2223x; 2.2143x over previous
import functools

import jax
import jax.numpy as jnp
from jax import lax
from jax.experimental import pallas as pl
from jax.experimental.pallas import tpu as pltpu
from jax.experimental.pallas import tpu_sc as plsc

_L = 16


def _fast_rsqrt(x):
    i = lax.bitcast_convert_type(x, jnp.int32)
    i = jnp.int32(0x5F3759DF) - lax.shift_right_arithmetic(i, 1)
    y = lax.bitcast_convert_type(i, jnp.float32)
    for _ in range(2):
        y = y * (1.5 - 0.5 * x * y * y)
    return y


def _make_sc_kernel(B, T, C, K):
    assert C % _L == 0
    n_groups = T // _L
    c_chunks = C // _L
    mesh = plsc.VectorSubcoreMesh(
        core_axis_name="c", subcore_axis_name="s", num_cores=2, num_subcores=16
    )

    @functools.partial(
        pl.kernel,
        out_type=jax.ShapeDtypeStruct((B, C), jnp.float32),
        mesh=mesh,
        scratch_types=[
            pltpu.VMEM((T, C), jnp.float32),
            pltpu.VMEM((K, T), jnp.int32),
            pltpu.VMEM((T, _L), jnp.int32),
            pltpu.VMEM((T,), jnp.float32),
            pltpu.VMEM((T,), jnp.float32),
            pltpu.VMEM((K * _L, _L + 1), jnp.float32),
            pltpu.VMEM((C,), jnp.float32),
        ],
        compiler_params=pltpu.CompilerParams(
            needs_layout_passes=False, use_tc_tiling_on_sc=False
        ),
    )
    def sc_kernel(
        x_hbm, idxt_hbm, idxp_hbm, out_hbm,
        x_v, idx_v, idxp_v, n2_v, w_v, red_v, o_v,
    ):
        cid = lax.axis_index("c")
        sid = lax.axis_index("s")
        b = sid * 2 + cid

        pltpu.sync_copy(x_hbm.at[b], x_v)
        pltpu.sync_copy(idxt_hbm, idx_v)
        pltpu.sync_copy(idxp_hbm, idxp_v)
        plsc.subcore_barrier()

        zero16 = jnp.zeros((_L,), jnp.float32)
        lane = lax.iota(jnp.int32, _L)

        def norm_body(g, carry):
            @plsc.parallel_loop(0, _L, unroll=2)
            def _(sl):
                s = g * _L + sl
                acc = x_v[s, pl.ds(0, _L)] * x_v[s, pl.ds(0, _L)]
                for k in range(1, c_chunks):
                    v = x_v[s, pl.ds(k * _L, _L)]
                    acc = acc + v * v
                red_v[sl, pl.ds(0, _L)] = acc
            n2 = zero16
            for c in range(_L):
                n2 = n2 + plsc.load_gather(
                    red_v, [lane, jnp.full((_L,), c, jnp.int32)]
                )
            n2_v[pl.ds(g * _L, _L)] = n2
            w_v[pl.ds(g * _L, _L)] = zero16
            return carry

        lax.fori_loop(0, n_groups, norm_body, 0)
        plsc.subcore_barrier()

        def group_body(g, carry):
            t0 = g * _L

            @plsc.parallel_loop(0, _L, unroll=1)
            def _(tl):
                t = t0 + tl
                q = [x_v[t, pl.ds(k * _L, _L)] for k in range(c_chunks)]
                idxrow = idxp_v[t, pl.ds(0, _L)]
                for j in range(K):
                    s = idxrow[j]
                    acc = q[0] * x_v[s, pl.ds(0, _L)]
                    for k in range(1, c_chunks):
                        acc = acc + q[k] * x_v[s, pl.ds(k * _L, _L)]
                    red_v[j * _L + tl, pl.ds(0, _L)] = acc

            idxv = [idx_v[j, pl.ds(t0, _L)] for j in range(K)]

            dots = []
            for j in range(K):
                rows = j * _L + lane
                d = plsc.load_gather(red_v, [rows, jnp.zeros((_L,), jnp.int32)])
                for c in range(1, _L):
                    d = d + plsc.load_gather(
                        red_v, [rows, jnp.full((_L,), c, jnp.int32)]
                    )
                dots.append(d)

            q2 = n2_v[pl.ds(t0, _L)]
            n2s = [plsc.load_gather(n2_v, [idxv[j]]) for j in range(K)]
            sims = [
                dots[j] * _fast_rsqrt(jnp.maximum(q2 * n2s[j], 1e-16))
                for j in range(K)
            ]
            m = sims[0]
            for j in range(1, K):
                m = jnp.maximum(m, sims[j])
            exps = [jnp.exp(s - m) for s in sims]
            tot = exps[0]
            for j in range(1, K):
                tot = tot + exps[j]
            r = 1.0 / tot
            for j in range(K):
                plsc.addupdate_scatter(w_v, [idxv[j]], exps[j] * r)
            return carry

        lax.fori_loop(0, n_groups, group_body, 0)

        @plsc.parallel_loop(
            0, n_groups, unroll=1,
            carry=tuple(zero16 for _ in range(c_chunks)),
        )
        def acc(g, a):
            w16 = w_v[pl.ds(g * _L, _L)]
            for sl in range(_L):
                s = g * _L + sl
                wv = jnp.full((_L,), w16[sl])
                a = tuple(
                    a[k] + wv * x_v[s, pl.ds(k * _L, _L)]
                    for k in range(c_chunks)
                )
            return a
        scale = jnp.float32(1.0 / T)
        for k in range(c_chunks):
            o_v[pl.ds(k * _L, _L)] = acc[k] * scale
        pltpu.sync_copy(o_v, out_hbm.at[b])

    return sc_kernel


def kernel(X, neighbor_idx):
    B, T, C = X.shape
    K = neighbor_idx.shape[1]
    idx32 = neighbor_idx.astype(jnp.int32)
    idx_pad = jnp.pad(idx32, ((0, 0), (0, _L - K)))
    out = _make_sc_kernel(B, T, C, K)(X, idx32.T, idx_pad)
    return out.reshape(B, 1, C)


# speedup vs baseline: 2.2546x; 1.0145x over previous
import functools

import jax
import jax.numpy as jnp
from jax import lax
from jax.experimental import pallas as pl
from jax.experimental.pallas import tpu as pltpu
from jax.experimental.pallas import tpu_sc as plsc

_L = 16


def _fast_rsqrt(x):
    i = lax.bitcast_convert_type(x, jnp.int32)
    i = jnp.int32(0x5F3759DF) - lax.shift_right_arithmetic(i, 1)
    y = lax.bitcast_convert_type(i, jnp.float32)
    for _ in range(2):
        y = y * (1.5 - 0.5 * x * y * y)
    return y


def _make_sc_kernel(B, T, C, K):
    assert C % _L == 0
    n_groups = T // _L
    c_chunks = C // _L
    mesh = plsc.VectorSubcoreMesh(
        core_axis_name="c", subcore_axis_name="s", num_cores=2, num_subcores=16
    )

    @functools.partial(
        pl.kernel,
        out_type=jax.ShapeDtypeStruct((B, C), jnp.float32),
        mesh=mesh,
        scratch_types=[
            pltpu.VMEM((T, C), jnp.float32),
            pltpu.VMEM((K, T), jnp.int32),
            pltpu.VMEM((T, _L), jnp.int32),
            pltpu.VMEM((T,), jnp.float32),
            pltpu.VMEM((T,), jnp.float32),
            pltpu.VMEM((K * _L, _L + 1), jnp.float32),
            pltpu.VMEM((C,), jnp.float32),
        ],
        compiler_params=pltpu.CompilerParams(
            needs_layout_passes=False, use_tc_tiling_on_sc=False
        ),
    )
    def sc_kernel(
        x_hbm, idxt_hbm, idxp_hbm, out_hbm,
        x_v, idx_v, idxp_v, n2_v, w_v, red_v, o_v,
    ):
        cid = lax.axis_index("c")
        sid = lax.axis_index("s")
        b = sid * 2 + cid

        pltpu.sync_copy(x_hbm.at[b], x_v)
        pltpu.sync_copy(idxt_hbm, idx_v)
        pltpu.sync_copy(idxp_hbm, idxp_v)
        plsc.subcore_barrier()

        zero16 = jnp.zeros((_L,), jnp.float32)
        lane = lax.iota(jnp.int32, _L)

        def norm_body(g, carry):
            @plsc.parallel_loop(0, _L, unroll=2)
            def _(sl):
                s = g * _L + sl
                acc = x_v[s, pl.ds(0, _L)] * x_v[s, pl.ds(0, _L)]
                for k in range(1, c_chunks):
                    v = x_v[s, pl.ds(k * _L, _L)]
                    acc = acc + v * v
                red_v[sl, pl.ds(0, _L)] = acc
            n2 = zero16
            for c in range(_L):
                n2 = n2 + plsc.load_gather(
                    red_v, [lane, jnp.full((_L,), c, jnp.int32)]
                )
            n2_v[pl.ds(g * _L, _L)] = n2
            w_v[pl.ds(g * _L, _L)] = zero16
            return carry

        lax.fori_loop(0, n_groups, norm_body, 0)
        plsc.subcore_barrier()

        def group_body(g, carry):
            t0 = g * _L

            @plsc.parallel_loop(0, _L, unroll=2)
            def _(tl):
                t = t0 + tl
                q = [x_v[t, pl.ds(k * _L, _L)] for k in range(c_chunks)]
                idxrow = idxp_v[t, pl.ds(0, _L)]
                for j in range(K):
                    s = idxrow[j]
                    acc = q[0] * x_v[s, pl.ds(0, _L)]
                    for k in range(1, c_chunks):
                        acc = acc + q[k] * x_v[s, pl.ds(k * _L, _L)]
                    red_v[j * _L + tl, pl.ds(0, _L)] = acc

            idxv = [idx_v[j, pl.ds(t0, _L)] for j in range(K)]

            dots = []
            for j in range(K):
                rows = j * _L + lane
                d = plsc.load_gather(red_v, [rows, jnp.zeros((_L,), jnp.int32)])
                for c in range(1, _L):
                    d = d + plsc.load_gather(
                        red_v, [rows, jnp.full((_L,), c, jnp.int32)]
                    )
                dots.append(d)

            q2 = n2_v[pl.ds(t0, _L)]
            n2s = [plsc.load_gather(n2_v, [idxv[j]]) for j in range(K)]
            sims = [
                dots[j] * _fast_rsqrt(jnp.maximum(q2 * n2s[j], 1e-16))
                for j in range(K)
            ]
            m = sims[0]
            for j in range(1, K):
                m = jnp.maximum(m, sims[j])
            exps = [jnp.exp(s - m) for s in sims]
            tot = exps[0]
            for j in range(1, K):
                tot = tot + exps[j]
            r = 1.0 / tot
            for j in range(K):
                plsc.addupdate_scatter(w_v, [idxv[j]], exps[j] * r)
            return carry

        lax.fori_loop(0, n_groups, group_body, 0)

        @plsc.parallel_loop(
            0, n_groups, unroll=1,
            carry=tuple(zero16 for _ in range(c_chunks)),
        )
        def acc(g, a):
            w16 = w_v[pl.ds(g * _L, _L)]
            for sl in range(_L):
                s = g * _L + sl
                wv = jnp.full((_L,), w16[sl])
                a = tuple(
                    a[k] + wv * x_v[s, pl.ds(k * _L, _L)]
                    for k in range(c_chunks)
                )
            return a
        scale = jnp.float32(1.0 / T)
        for k in range(c_chunks):
            o_v[pl.ds(k * _L, _L)] = acc[k] * scale
        pltpu.sync_copy(o_v, out_hbm.at[b])

    return sc_kernel


def kernel(X, neighbor_idx):
    B, T, C = X.shape
    K = neighbor_idx.shape[1]
    idx32 = neighbor_idx.astype(jnp.int32)
    idx_pad = jnp.pad(idx32, ((0, 0), (0, _L - K)))
    out = _make_sc_kernel(B, T, C, K)(X, idx32.T, idx_pad)
    return out.reshape(B, 1, C)


# speedup vs baseline: 2.2886x; 1.0151x over previous
import functools

import jax
import jax.numpy as jnp
from jax import lax
from jax.experimental import pallas as pl
from jax.experimental.pallas import tpu as pltpu
from jax.experimental.pallas import tpu_sc as plsc

_L = 16


def _fast_rsqrt(x):
    i = lax.bitcast_convert_type(x, jnp.int32)
    i = jnp.int32(0x5F3759DF) - lax.shift_right_arithmetic(i, 1)
    y = lax.bitcast_convert_type(i, jnp.float32)
    for _ in range(2):
        y = y * (1.5 - 0.5 * x * y * y)
    return y


def _make_sc_kernel(B, T, C, K):
    assert C % _L == 0
    n_groups = T // _L
    c_chunks = C // _L
    mesh = plsc.VectorSubcoreMesh(
        core_axis_name="c", subcore_axis_name="s", num_cores=2, num_subcores=16
    )

    @functools.partial(
        pl.kernel,
        out_type=jax.ShapeDtypeStruct((B, C), jnp.float32),
        mesh=mesh,
        scratch_types=[
            pltpu.VMEM((T, C), jnp.float32),
            pltpu.VMEM((K, T), jnp.int32),
            pltpu.VMEM((T, _L), jnp.int32),
            pltpu.VMEM((T,), jnp.float32),
            pltpu.VMEM((T,), jnp.float32),
            pltpu.VMEM((K * _L, _L + 1), jnp.float32),
            pltpu.VMEM((K, _L + 1), jnp.float32),
            pltpu.VMEM((C,), jnp.float32),
        ],
        compiler_params=pltpu.CompilerParams(
            needs_layout_passes=False, use_tc_tiling_on_sc=False
        ),
    )
    def sc_kernel(
        x_hbm, idxt_hbm, idxp_hbm, out_hbm,
        x_v, idx_v, idxp_v, n2_v, w_v, red_v, sims_v, o_v,
    ):
        cid = lax.axis_index("c")
        sid = lax.axis_index("s")
        b = sid * 2 + cid

        pltpu.sync_copy(x_hbm.at[b], x_v)
        pltpu.sync_copy(idxt_hbm, idx_v)
        pltpu.sync_copy(idxp_hbm, idxp_v)
        plsc.subcore_barrier()

        zero16 = jnp.zeros((_L,), jnp.float32)
        lane = lax.iota(jnp.int32, _L)

        def norm_body(g, carry):
            @plsc.parallel_loop(0, _L, unroll=2)
            def _(sl):
                s = g * _L + sl
                acc = x_v[s, pl.ds(0, _L)] * x_v[s, pl.ds(0, _L)]
                for k in range(1, c_chunks):
                    v = x_v[s, pl.ds(k * _L, _L)]
                    acc = acc + v * v
                red_v[sl, pl.ds(0, _L)] = acc
            n2 = zero16
            for c in range(_L):
                n2 = n2 + plsc.load_gather(
                    red_v, [lane, jnp.full((_L,), c, jnp.int32)]
                )
            n2_v[pl.ds(g * _L, _L)] = n2
            w_v[pl.ds(g * _L, _L)] = zero16
            return carry

        lax.fori_loop(0, n_groups, norm_body, 0)
        plsc.subcore_barrier()

        def group_body(g, carry):
            t0 = g * _L

            @plsc.parallel_loop(0, _L, unroll=2)
            def _(tl):
                t = t0 + tl
                q = [x_v[t, pl.ds(k * _L, _L)] for k in range(c_chunks)]
                idxrow = idxp_v[t, pl.ds(0, _L)]
                for j in range(K):
                    s = idxrow[j]
                    acc = q[0] * x_v[s, pl.ds(0, _L)]
                    for k in range(1, c_chunks):
                        acc = acc + q[k] * x_v[s, pl.ds(k * _L, _L)]
                    red_v[j * _L + tl, pl.ds(0, _L)] = acc

            idxv = [idx_v[j, pl.ds(t0, _L)] for j in range(K)]
            q2 = n2_v[pl.ds(t0, _L)]

            @plsc.parallel_loop(0, K, unroll=1)
            def _(j):
                rows = j * _L + lane
                d = plsc.load_gather(red_v, [rows, jnp.zeros((_L,), jnp.int32)])
                for c in range(1, _L):
                    d = d + plsc.load_gather(
                        red_v, [rows, jnp.full((_L,), c, jnp.int32)]
                    )
                idxj = plsc.load_gather(idx_v, [jnp.full((_L,), j), t0 + lane])
                n2j = plsc.load_gather(n2_v, [idxj])
                sims_v[j, pl.ds(0, _L)] = d * _fast_rsqrt(
                    jnp.maximum(q2 * n2j, 1e-16)
                )

            sims = [sims_v[j, pl.ds(0, _L)] for j in range(K)]
            m = sims[0]
            for j in range(1, K):
                m = jnp.maximum(m, sims[j])
            exps = [jnp.exp(s - m) for s in sims]
            tot = exps[0]
            for j in range(1, K):
                tot = tot + exps[j]
            r = 1.0 / tot
            for j in range(K):
                plsc.addupdate_scatter(w_v, [idxv[j]], exps[j] * r)
            return carry

        lax.fori_loop(0, n_groups, group_body, 0)

        @plsc.parallel_loop(
            0, n_groups, unroll=1,
            carry=tuple(zero16 for _ in range(c_chunks)),
        )
        def acc(g, a):
            w16 = w_v[pl.ds(g * _L, _L)]
            for sl in range(_L):
                s = g * _L + sl
                wv = jnp.full((_L,), w16[sl])
                a = tuple(
                    a[k] + wv * x_v[s, pl.ds(k * _L, _L)]
                    for k in range(c_chunks)
                )
            return a
        scale = jnp.float32(1.0 / T)
        for k in range(c_chunks):
            o_v[pl.ds(k * _L, _L)] = acc[k] * scale
        pltpu.sync_copy(o_v, out_hbm.at[b])

    return sc_kernel


def kernel(X, neighbor_idx):
    B, T, C = X.shape
    K = neighbor_idx.shape[1]
    idx32 = neighbor_idx.astype(jnp.int32)
    idx_pad = jnp.pad(idx32, ((0, 0), (0, _L - K)))
    out = _make_sc_kernel(B, T, C, K)(X, idx32.T, idx_pad)
    return out.reshape(B, 1, C)
